# baseline (speedup 1.0000x reference)
import functools

import jax
import jax.numpy as jnp
from jax import lax
from jax.experimental import pallas as pl
from jax.experimental.pallas import tpu as pltpu
from jax.experimental.pallas import tpu_sc as plsc

B = 16384
D = 32
O = 64
H = 64

NC = 2
NS = 16
NW = NC * NS
B_PER_W = B // NW
CH = 128
NCHUNK = B_PER_W // CH


def _sc_gather_body(uidx, midx, utab, mtab, uout, mout, idx_v, rows_v, sem):
    wid = lax.axis_index("s") * NC + lax.axis_index("c")
    base = wid * B_PER_W
    copies = []
    for t, (idx_hbm, tab_hbm) in enumerate(((uidx, utab), (midx, mtab))):
        for j in range(NCHUNK):
            r = t * NCHUNK + j
            pltpu.sync_copy(idx_hbm.at[pl.ds(base + j * CH, CH)], idx_v.at[r])
            copies.append(pltpu.async_copy(tab_hbm.at[idx_v.at[r]], rows_v.at[r], sem))
    for c in copies:
        c.wait()
    for t, out_hbm in enumerate((uout, mout)):
        for j in range(NCHUNK):
            r = t * NCHUNK + j
            pltpu.sync_copy(rows_v.at[r], out_hbm.at[pl.ds(base + j * CH, CH)])


@functools.partial(jax.jit, static_argnames=())
def _sc_gather(user_inp, movie_inp, user_table, movie_table):
    mesh = plsc.VectorSubcoreMesh(
        core_axis_name="c", subcore_axis_name="s", num_cores=NC, num_subcores=NS
    )
    return pl.kernel(
        _sc_gather_body,
        out_type=(
            jax.ShapeDtypeStruct((B, D), jnp.float32),
            jax.ShapeDtypeStruct((B, D), jnp.float32),
        ),
        mesh=mesh,
        scratch_types=[
            pltpu.VMEM((2 * NCHUNK, CH), jnp.int32),
            pltpu.VMEM((2 * NCHUNK, CH, D), jnp.float32),
            pltpu.SemaphoreType.DMA,
        ],
        compiler_params=pltpu.CompilerParams(use_tc_tiling_on_sc=False),
    )(user_inp, movie_inp, user_table, movie_table)


def _tc_dense_body(ug, mg, oth, w_h, b_h, w_o, b_o, out):
    z = jnp.dot(oth[...], w_h[...], preferred_element_type=jnp.float32) + b_h[...]
    a = jnp.where(z >= 0, z, 0.01 * z)
    r = (
        jnp.dot(ug[...], w_o[0:D, :], preferred_element_type=jnp.float32)
        + jnp.dot(mg[...], w_o[D:2 * D, :], preferred_element_type=jnp.float32)
        + jnp.dot(a, w_o[2 * D:, :], preferred_element_type=jnp.float32)
        + b_o[...]
    )
    out[...] = r[:, 0]


def kernel(user_inp, movie_inp, others_inp, user_table, movie_table, W_h, b_h, W_o, b_o):
    ug, mg = _sc_gather(
        user_inp.astype(jnp.int32), movie_inp.astype(jnp.int32), user_table, movie_table
    )
    out = pl.pallas_call(
        _tc_dense_body,
        out_shape=jax.ShapeDtypeStruct((B,), jnp.float32),
    )(ug, mg, others_inp, W_h, b_h, W_o, b_o)
    return out
